# baseline (speedup 1.0000x reference)
import jax
import jax.numpy as jnp
from jax import lax
from jax.experimental import pallas as pl
from jax.experimental.pallas import tpu as pltpu
from jax.experimental.pallas import tpu_sc as plsc

N_NODES = 10000
N_EDGES = 320000
D_FEAT = 128

NC = 2
NS = 16
NW = NC * NS

N_PAD = 10240
CHUNK = 128
N_CHUNKS = 80
E_PER_W = CHUNK * N_CHUNKS
E_PAD = NW * E_PER_W
ROWS_PER_TILE = N_PAD // NS
ZROWS = 16


def _sc_aggregate_body(x_hbm, ra_hbm, rb_hbm, out_hbm,
                       ia0, ib0, ia1, ib1, rows0, rows1, zbuf, acc,
                       si0, si1, sr0, sr1):
    cid = lax.axis_index("c")
    sid = lax.axis_index("s")
    wid = cid * NS + sid
    ebase = wid * E_PER_W

    def idx_start(i, ia, ib, s):
        base = ebase + i * CHUNK
        pltpu.async_copy(ra_hbm.at[pl.ds(base, CHUNK)], ia, s)
        pltpu.async_copy(rb_hbm.at[pl.ds(base, CHUNK)], ib, s)

    def idx_wait(ia, ib, s):
        pltpu.make_async_copy(ra_hbm.at[pl.ds(0, CHUNK)], ia, s).wait()
        pltpu.make_async_copy(rb_hbm.at[pl.ds(0, CHUNK)], ib, s).wait()

    def rows_wait(rows, s):
        pltpu.make_async_copy(x_hbm.at[pl.ds(0, CHUNK)], rows, s).wait()

    idx_start(0, ia0, ib0, si0)
    idx_start(1, ia1, ib1, si1)

    def fill_zero(i, _):
        r = i // 8
        j = i % 8
        zbuf[r, pl.ds(j * 16, 16)] = jnp.zeros((16,), jnp.float32)
        return 0

    lax.fori_loop(0, ZROWS * 8, fill_zero, 0)

    row0 = sid * ROWS_PER_TILE

    def zero_acc(k, _):
        pltpu.sync_copy(zbuf, acc.at[pl.ds(row0 + k * ZROWS, ZROWS)])
        return 0

    lax.fori_loop(0, ROWS_PER_TILE // ZROWS, zero_acc, 0)

    plsc.subcore_barrier()

    idx_wait(ia0, ib0, si0)
    pltpu.async_copy(x_hbm.at[ia0], rows0, sr0)

    def pair(k, _):
        i0 = 2 * k
        idx_wait(ia1, ib1, si1)
        pltpu.async_copy(x_hbm.at[ia1], rows1, sr1)
        rows_wait(rows0, sr0)
        pltpu.sync_copy(rows0, acc.at[ib0], add=True)

        @pl.when(i0 + 2 < N_CHUNKS)
        def _():
            idx_start(i0 + 2, ia0, ib0, si0)
            idx_wait(ia0, ib0, si0)
            pltpu.async_copy(x_hbm.at[ia0], rows0, sr0)

        rows_wait(rows1, sr1)
        pltpu.sync_copy(rows1, acc.at[ib1], add=True)

        @pl.when(i0 + 3 < N_CHUNKS)
        def _():
            idx_start(i0 + 3, ia1, ib1, si1)

        return 0

    lax.fori_loop(0, N_CHUNKS // 2, pair, 0)

    plsc.subcore_barrier()

    obase = cid * N_PAD + row0
    pltpu.sync_copy(acc.at[pl.ds(row0, ROWS_PER_TILE)],
                    out_hbm.at[pl.ds(obase, ROWS_PER_TILE)])


def _sc_aggregate(X, ref_a, ref_b):
    mesh = plsc.VectorSubcoreMesh(core_axis_name="c", subcore_axis_name="s",
                                  num_cores=NC, num_subcores=NS)
    f = pl.kernel(
        _sc_aggregate_body,
        out_type=jax.ShapeDtypeStruct((NC * N_PAD, D_FEAT), jnp.float32),
        mesh=mesh,
        scratch_types=[
            pltpu.VMEM((CHUNK,), jnp.int32),
            pltpu.VMEM((CHUNK,), jnp.int32),
            pltpu.VMEM((CHUNK,), jnp.int32),
            pltpu.VMEM((CHUNK,), jnp.int32),
            pltpu.VMEM((CHUNK, D_FEAT), jnp.float32),
            pltpu.VMEM((CHUNK, D_FEAT), jnp.float32),
            pltpu.VMEM((ZROWS, D_FEAT), jnp.float32),
            pltpu.VMEM_SHARED((N_PAD, D_FEAT), jnp.float32),
            pltpu.SemaphoreType.DMA,
            pltpu.SemaphoreType.DMA,
            pltpu.SemaphoreType.DMA,
            pltpu.SemaphoreType.DMA,
        ],
    )
    npad = E_PAD - N_EDGES
    ra = jnp.concatenate([ref_a, jnp.zeros((npad,), jnp.int32)])
    rb = jnp.concatenate([ref_b, jnp.full((npad,), N_NODES, jnp.int32)])
    return f(X, ra, rb)


def _mlp_body(x_ref, a0_ref, a1_ref, wh_ref, bh_ref, wo_ref, bo_ref, o_ref):
    xa = x_ref[...] + a0_ref[0] + a1_ref[0]
    h = jnp.dot(xa, wh_ref[...], preferred_element_type=jnp.float32)
    h = jnp.maximum(h + bh_ref[...], 0.0)
    o = jnp.dot(h, wo_ref[...], preferred_element_type=jnp.float32)
    o_ref[...] = jnp.maximum(o + bo_ref[...], 0.0)


def _mlp(X, agg3, W_hidden, b_hidden, W_out, b_out):
    R = 1000
    full = lambda i: (0, 0)
    return pl.pallas_call(
        _mlp_body,
        grid=(N_NODES // R,),
        in_specs=[
            pl.BlockSpec((R, D_FEAT), lambda i: (i, 0)),
            pl.BlockSpec((1, R, D_FEAT), lambda i: (0, i, 0)),
            pl.BlockSpec((1, R, D_FEAT), lambda i: (1, i, 0)),
            pl.BlockSpec((D_FEAT, D_FEAT), full),
            pl.BlockSpec((1, D_FEAT), full),
            pl.BlockSpec((D_FEAT, D_FEAT), full),
            pl.BlockSpec((1, D_FEAT), full),
        ],
        out_specs=pl.BlockSpec((R, D_FEAT), lambda i: (i, 0)),
        out_shape=jax.ShapeDtypeStruct((N_NODES, D_FEAT), jnp.float32),
    )(X, agg3, agg3, W_hidden, b_hidden, W_out, b_out)


@jax.jit
def kernel(X, ref_a, ref_b, W_hidden, b_hidden, W_out, b_out):
    ref_a = ref_a.astype(jnp.int32)
    ref_b = ref_b.astype(jnp.int32)
    agg = _sc_aggregate(X, ref_a, ref_b)
    agg3 = agg.reshape(NC, N_PAD, D_FEAT)
    return _mlp(X, agg3, W_hidden, b_hidden.reshape(1, -1),
                W_out, b_out.reshape(1, -1))


# speedup vs baseline: 1.0148x; 1.0148x over previous
import jax
import jax.numpy as jnp
from jax import lax
from jax.experimental import pallas as pl
from jax.experimental.pallas import tpu as pltpu
from jax.experimental.pallas import tpu_sc as plsc

N_NODES = 10000
N_EDGES = 320000
D_FEAT = 128

NC = 2
NS = 16
NW = NC * NS

N_PAD = 10240
CHUNK = 128
N_CHUNKS = 80
E_PER_W = CHUNK * N_CHUNKS
E_PAD = NW * E_PER_W
ROWS_PER_TILE = N_PAD // NS
ZROWS = 16


def _sc_aggregate_body(x_hbm, ra_hbm, rb_hbm, out_hbm,
                       ia0, ib0, ia1, ib1, ia2, ib2, ia3, ib3,
                       rows0, rows1, zbuf, acc,
                       si0, si1, si2, si3, sr0, sr1):
    cid = lax.axis_index("c")
    sid = lax.axis_index("s")
    wid = cid * NS + sid
    ebase = wid * E_PER_W

    ias = (ia0, ia1, ia2, ia3)
    ibs = (ib0, ib1, ib2, ib3)
    sis = (si0, si1, si2, si3)
    rws = (rows0, rows1)
    srs = (sr0, sr1)

    def idx_start(i, p):
        base = ebase + i * CHUNK
        pltpu.async_copy(ra_hbm.at[pl.ds(base, CHUNK)], ias[p], sis[p])
        pltpu.async_copy(rb_hbm.at[pl.ds(base, CHUNK)], ibs[p], sis[p])

    def idx_wait(p):
        pltpu.make_async_copy(ra_hbm.at[pl.ds(0, CHUNK)], ias[p], sis[p]).wait()
        pltpu.make_async_copy(rb_hbm.at[pl.ds(0, CHUNK)], ibs[p], sis[p]).wait()

    def rows_wait(p):
        pltpu.make_async_copy(x_hbm.at[pl.ds(0, CHUNK)], rws[p], srs[p]).wait()

    for p in range(4):
        idx_start(p, p)

    def fill_zero(i, _):
        r = i // 8
        j = i % 8
        zbuf[r, pl.ds(j * 16, 16)] = jnp.zeros((16,), jnp.float32)
        return 0

    lax.fori_loop(0, ZROWS * 8, fill_zero, 0)

    row0 = sid * ROWS_PER_TILE

    def zero_acc(k, _):
        pltpu.sync_copy(zbuf, acc.at[pl.ds(row0 + k * ZROWS, ZROWS)])
        return 0

    lax.fori_loop(0, ROWS_PER_TILE // ZROWS, zero_acc, 0)

    plsc.subcore_barrier()

    idx_wait(0)
    pltpu.async_copy(x_hbm.at[ia0], rws[0], srs[0])

    def quad(k, _):
        for u in range(4):
            i = 4 * k + u
            pq = (u + 1) % 4
            pr = u % 2

            @pl.when(i + 1 < N_CHUNKS)
            def _(pq=pq, pr=pr):
                idx_wait(pq)
                pltpu.async_copy(x_hbm.at[ias[pq]], rws[1 - pr], srs[1 - pr])

            rows_wait(pr)
            pltpu.sync_copy(rws[pr], acc.at[ibs[u]], add=True)

            @pl.when(i + 4 < N_CHUNKS)
            def _(u=u, i=i):
                idx_start(i + 4, u)

        return 0

    lax.fori_loop(0, N_CHUNKS // 4, quad, 0)

    plsc.subcore_barrier()

    obase = cid * N_PAD + row0
    pltpu.sync_copy(acc.at[pl.ds(row0, ROWS_PER_TILE)],
                    out_hbm.at[pl.ds(obase, ROWS_PER_TILE)])


def _sc_aggregate(X, ref_a, ref_b):
    mesh = plsc.VectorSubcoreMesh(core_axis_name="c", subcore_axis_name="s",
                                  num_cores=NC, num_subcores=NS)
    f = pl.kernel(
        _sc_aggregate_body,
        out_type=jax.ShapeDtypeStruct((NC * N_PAD, D_FEAT), jnp.float32),
        mesh=mesh,
        scratch_types=(
            [pltpu.VMEM((CHUNK,), jnp.int32)] * 8
            + [pltpu.VMEM((CHUNK, D_FEAT), jnp.float32)] * 2
            + [pltpu.VMEM((ZROWS, D_FEAT), jnp.float32),
               pltpu.VMEM_SHARED((N_PAD, D_FEAT), jnp.float32)]
            + [pltpu.SemaphoreType.DMA] * 6
        ),
    )
    npad = E_PAD - N_EDGES
    ra = jnp.concatenate([ref_a, jnp.zeros((npad,), jnp.int32)])
    rb = jnp.concatenate([ref_b, jnp.full((npad,), N_NODES, jnp.int32)])
    return f(X, ra, rb)


def _mlp_body(x_ref, a0_ref, a1_ref, wh_ref, bh_ref, wo_ref, bo_ref, o_ref):
    xa = x_ref[...] + a0_ref[0] + a1_ref[0]
    h = jnp.dot(xa, wh_ref[...], preferred_element_type=jnp.float32)
    h = jnp.maximum(h + bh_ref[...], 0.0)
    o = jnp.dot(h, wo_ref[...], preferred_element_type=jnp.float32)
    o_ref[...] = jnp.maximum(o + bo_ref[...], 0.0)


def _mlp(X, agg3, W_hidden, b_hidden, W_out, b_out):
    R = 1000
    full = lambda i: (0, 0)
    return pl.pallas_call(
        _mlp_body,
        grid=(N_NODES // R,),
        in_specs=[
            pl.BlockSpec((R, D_FEAT), lambda i: (i, 0)),
            pl.BlockSpec((1, R, D_FEAT), lambda i: (0, i, 0)),
            pl.BlockSpec((1, R, D_FEAT), lambda i: (1, i, 0)),
            pl.BlockSpec((D_FEAT, D_FEAT), full),
            pl.BlockSpec((1, D_FEAT), full),
            pl.BlockSpec((D_FEAT, D_FEAT), full),
            pl.BlockSpec((1, D_FEAT), full),
        ],
        out_specs=pl.BlockSpec((R, D_FEAT), lambda i: (i, 0)),
        out_shape=jax.ShapeDtypeStruct((N_NODES, D_FEAT), jnp.float32),
    )(X, agg3, agg3, W_hidden, b_hidden, W_out, b_out)


@jax.jit
def kernel(X, ref_a, ref_b, W_hidden, b_hidden, W_out, b_out):
    ref_a = ref_a.astype(jnp.int32)
    ref_b = ref_b.astype(jnp.int32)
    agg = _sc_aggregate(X, ref_a, ref_b)
    agg3 = agg.reshape(NC, N_PAD, D_FEAT)
    return _mlp(X, agg3, W_hidden, b_hidden.reshape(1, -1),
                W_out, b_out.reshape(1, -1))
